# Initial kernel scaffold; baseline (speedup 1.0000x reference)
import jax
import jax.numpy as jnp
from jax.experimental import pallas as pl


def kernel(utype, hp, utype_table, hp_table):
    raise NotImplementedError("write your pallas kernel here")


# speedup vs baseline: 2.3178x; 2.3178x over previous
import functools

import jax
import jax.numpy as jnp
from jax import lax
from jax.experimental import pallas as pl
from jax.experimental.pallas import tpu as pltpu
from jax.experimental.pallas import tpu_sc as plsc

EMB_DIM = 32
NUM_HP_BINS = 256

NC = 2
NS = 16
NW = NC * NS

CHUNK = 1024


def _make_kernel(n_total, n_utype):
  per_w = n_total // NW
  n_chunks = per_w // CHUNK
  n_groups = CHUNK // 16
  mesh = plsc.VectorSubcoreMesh(
      core_axis_name="c", subcore_axis_name="s", num_cores=NC, num_subcores=NS
  )

  @functools.partial(
      pl.kernel,
      out_type=jax.ShapeDtypeStruct((n_total * 2 * EMB_DIM,), jnp.float32),
      mesh=mesh,
      scratch_types=[
          pltpu.VMEM((n_utype * EMB_DIM,), jnp.float32),
          pltpu.VMEM((NUM_HP_BINS * EMB_DIM,), jnp.float32),
          pltpu.VMEM((CHUNK,), jnp.int32),
          pltpu.VMEM((CHUNK,), jnp.float32),
          pltpu.VMEM((CHUNK * 2 * EMB_DIM,), jnp.float32),
      ],
      compiler_params=pltpu.CompilerParams(
          use_tc_tiling_on_sc=False, needs_layout_passes=False),
  )
  def emb_kernel(ut_hbm, hp_hbm, utab_hbm, htab_hbm, out_hbm,
                 utab_v, htab_v, uidx_v, hp_v, cbuf):
    wid = lax.axis_index("s") * NC + lax.axis_index("c")
    base = wid * per_w

    pltpu.sync_copy(utab_hbm, utab_v)
    pltpu.sync_copy(htab_hbm, htab_v)

    lanes = lax.iota(jnp.int32, 16)

    def chunk_body(i, _):
      gbase = base + i * CHUNK
      pltpu.sync_copy(ut_hbm.at[pl.ds(gbase, CHUNK)], uidx_v)
      pltpu.sync_copy(hp_hbm.at[pl.ds(gbase, CHUNK)], hp_v)

      def group_body(k, _):
        u = uidx_v[pl.ds(k * 16, 16)]
        h = (hp_v[pl.ds(k * 16, 16)] * float(NUM_HP_BINS - 1)).astype(
            jnp.int32)
        ubase = u << 5
        hbase = h << 5
        obase = (k * 16 + lanes) << 6
        for j in range(EMB_DIM):
          vu = plsc.load_gather(utab_v, [ubase + j])
          plsc.store_scatter(cbuf, [obase + j], vu)
          vh = plsc.load_gather(htab_v, [hbase + j])
          plsc.store_scatter(cbuf, [obase + (EMB_DIM + j)], vh)
        return ()

      lax.fori_loop(0, n_groups, group_body, ())
      pltpu.sync_copy(cbuf, out_hbm.at[pl.ds(gbase * 2 * EMB_DIM,
                                             CHUNK * 2 * EMB_DIM)])
      return ()

    lax.fori_loop(0, n_chunks, chunk_body, ())

  return emb_kernel


def kernel(utype, hp, utype_table, hp_table):
  b, u = utype.shape
  n_total = b * u
  n_utype = utype_table.shape[0]
  ut_flat = utype.reshape(n_total).astype(jnp.int32)
  hp_flat = hp.reshape(n_total)
  out = _make_kernel(n_total, n_utype)(
      ut_flat, hp_flat,
      utype_table.reshape(n_utype * EMB_DIM),
      hp_table.reshape(NUM_HP_BINS * EMB_DIM))
  return out.reshape(b, u, 2 * EMB_DIM)


# speedup vs baseline: 4.9850x; 2.1508x over previous
import functools

import jax
import jax.numpy as jnp
from jax import lax
from jax.experimental import pallas as pl
from jax.experimental.pallas import tpu as pltpu
from jax.experimental.pallas import tpu_sc as plsc

EMB_DIM = 32
NUM_HP_BINS = 256

NC = 2
NS = 16
NW = NC * NS

CHUNK = 1024
TROW = EMB_DIM + 1
OROW = 2 * EMB_DIM + 1


def _make_kernel(n_total, n_utype):
  per_w = n_total // NW
  n_chunks = per_w // CHUNK
  n_groups = CHUNK // 16
  mesh = plsc.VectorSubcoreMesh(
      core_axis_name="c", subcore_axis_name="s", num_cores=NC, num_subcores=NS
  )

  @functools.partial(
      pl.kernel,
      out_type=jax.ShapeDtypeStruct((n_total, 2 * EMB_DIM), jnp.float32),
      mesh=mesh,
      scratch_types=[
          pltpu.VMEM((n_utype * TROW,), jnp.float32),
          pltpu.VMEM((NUM_HP_BINS * TROW,), jnp.float32),
          pltpu.VMEM((CHUNK,), jnp.int32),
          pltpu.VMEM((CHUNK,), jnp.float32),
          pltpu.VMEM((CHUNK, OROW), jnp.float32),
      ],
      compiler_params=pltpu.CompilerParams(
          use_tc_tiling_on_sc=False, needs_layout_passes=False),
  )
  def emb_kernel(ut_hbm, hp_hbm, utab_hbm, htab_hbm, out_hbm,
                 utab_v, htab_v, uidx_v, hp_v, cbuf):
    wid = lax.axis_index("s") * NC + lax.axis_index("c")
    base = wid * per_w

    pltpu.sync_copy(utab_hbm, utab_v)
    pltpu.sync_copy(htab_hbm, htab_v)

    lanes = lax.iota(jnp.int32, 16)

    def chunk_body(i, _):
      gbase = base + i * CHUNK
      pltpu.sync_copy(ut_hbm.at[pl.ds(gbase, CHUNK)], uidx_v)
      pltpu.sync_copy(hp_hbm.at[pl.ds(gbase, CHUNK)], hp_v)

      def group_body(k, _):
        u = uidx_v[pl.ds(k * 16, 16)]
        h = (hp_v[pl.ds(k * 16, 16)] * float(NUM_HP_BINS - 1)).astype(
            jnp.int32)
        ubase = u * TROW
        hbase = h * TROW
        orow = k * 16 + lanes
        for j in range(EMB_DIM):
          vu = plsc.load_gather(utab_v, [ubase + j])
          plsc.store_scatter(cbuf, [orow, jnp.full((16,), j, jnp.int32)], vu)
          vh = plsc.load_gather(htab_v, [hbase + j])
          plsc.store_scatter(
              cbuf, [orow, jnp.full((16,), EMB_DIM + j, jnp.int32)], vh)
        return ()

      lax.fori_loop(0, n_groups, group_body, ())
      pltpu.sync_copy(cbuf.at[:, pl.ds(0, 2 * EMB_DIM)],
                      out_hbm.at[pl.ds(gbase, CHUNK)])
      return ()

    lax.fori_loop(0, n_chunks, chunk_body, ())

  return emb_kernel


def kernel(utype, hp, utype_table, hp_table):
  b, u = utype.shape
  n_total = b * u
  n_utype = utype_table.shape[0]
  ut_flat = utype.reshape(n_total).astype(jnp.int32)
  hp_flat = hp.reshape(n_total)
  utab_p = jnp.pad(utype_table, ((0, 0), (0, TROW - EMB_DIM))).reshape(-1)
  htab_p = jnp.pad(hp_table, ((0, 0), (0, TROW - EMB_DIM))).reshape(-1)
  out = _make_kernel(n_total, n_utype)(ut_flat, hp_flat, utab_p, htab_p)
  return out.reshape(b, u, 2 * EMB_DIM)


# speedup vs baseline: 6.8011x; 1.3643x over previous
import functools

import jax
import jax.numpy as jnp
from jax import lax
from jax.experimental import pallas as pl
from jax.experimental.pallas import tpu as pltpu
from jax.experimental.pallas import tpu_sc as plsc

EMB_DIM = 32
NUM_HP_BINS = 256

NC = 2
NS = 16
NW = NC * NS

CHUNK = 1024
TROW = EMB_DIM + 1
OROW = 2 * EMB_DIM + 1


def _make_kernel(n_total, n_utype):
  per_w = n_total // NW
  n_chunks = per_w // CHUNK
  n_groups = CHUNK // 16
  mesh = plsc.VectorSubcoreMesh(
      core_axis_name="c", subcore_axis_name="s", num_cores=NC, num_subcores=NS
  )

  @functools.partial(
      pl.kernel,
      out_type=jax.ShapeDtypeStruct((n_total, 2 * EMB_DIM), jnp.float32),
      mesh=mesh,
      scratch_types=[
          pltpu.VMEM((n_utype * TROW,), jnp.float32),
          pltpu.VMEM((NUM_HP_BINS * TROW,), jnp.float32),
          pltpu.VMEM((CHUNK,), jnp.int32),
          pltpu.VMEM((CHUNK,), jnp.float32),
          pltpu.VMEM((CHUNK, OROW), jnp.float32),
      ],
      compiler_params=pltpu.CompilerParams(
          use_tc_tiling_on_sc=False, needs_layout_passes=False),
  )
  def emb_kernel(ut_hbm, hp_hbm, utab_hbm, htab_hbm, out_hbm,
                 utab_v, htab_v, uidx_v, hp_v, cbuf):
    wid = lax.axis_index("s") * NC + lax.axis_index("c")
    base = wid * per_w

    pltpu.sync_copy(utab_hbm, utab_v)
    pltpu.sync_copy(htab_hbm, htab_v)

    lanes = lax.iota(jnp.int32, 16)

    def chunk_body(i, _):
      gbase = base + i * CHUNK
      pltpu.sync_copy(ut_hbm.at[pl.ds(gbase, CHUNK)], uidx_v)
      pltpu.sync_copy(hp_hbm.at[pl.ds(gbase, CHUNK)], hp_v)

      @plsc.parallel_loop(0, n_groups, unroll=2)
      def group_body(k):
        u = uidx_v[pl.ds(k * 16, 16)]
        h = (hp_v[pl.ds(k * 16, 16)] * float(NUM_HP_BINS - 1)).astype(
            jnp.int32)
        ubase = u * TROW
        hbase = h * TROW
        orow = k * 16 + lanes
        for j in range(EMB_DIM):
          vu = plsc.load_gather(utab_v, [ubase + j])
          plsc.store_scatter(cbuf, [orow, jnp.full((16,), j, jnp.int32)], vu)
          vh = plsc.load_gather(htab_v, [hbase + j])
          plsc.store_scatter(
              cbuf, [orow, jnp.full((16,), EMB_DIM + j, jnp.int32)], vh)
      pltpu.sync_copy(cbuf.at[:, pl.ds(0, 2 * EMB_DIM)],
                      out_hbm.at[pl.ds(gbase, CHUNK)])
      return ()

    lax.fori_loop(0, n_chunks, chunk_body, ())

  return emb_kernel


def kernel(utype, hp, utype_table, hp_table):
  b, u = utype.shape
  n_total = b * u
  n_utype = utype_table.shape[0]
  ut_flat = utype.reshape(n_total).astype(jnp.int32)
  hp_flat = hp.reshape(n_total)
  utab_p = jnp.pad(utype_table, ((0, 0), (0, TROW - EMB_DIM))).reshape(-1)
  htab_p = jnp.pad(hp_table, ((0, 0), (0, TROW - EMB_DIM))).reshape(-1)
  out = _make_kernel(n_total, n_utype)(ut_flat, hp_flat, utab_p, htab_p)
  return out.reshape(b, u, 2 * EMB_DIM)


# speedup vs baseline: 10.4609x; 1.5381x over previous
import functools

import jax
import jax.numpy as jnp
from jax import lax
from jax.experimental import pallas as pl
from jax.experimental.pallas import tpu as pltpu
from jax.experimental.pallas import tpu_sc as plsc

EMB_DIM = 32
NUM_HP_BINS = 256

NC = 2
NS = 16
NW = NC * NS

BLOCK = 3200
ROWS = 2


def _make_kernel(b, u, n_utype):
  rows_w = b // NW
  n_blocks = (rows_w * u) // BLOCK
  rows_blk = BLOCK // u
  n_sub = rows_blk // ROWS
  mesh = plsc.VectorSubcoreMesh(
      core_axis_name="c", subcore_axis_name="s", num_cores=NC, num_subcores=NS
  )

  @functools.partial(
      pl.kernel,
      out_type=jax.ShapeDtypeStruct((b, u, 2 * EMB_DIM), jnp.float32),
      mesh=mesh,
      scratch_types=[
          pltpu.VMEM((n_utype * EMB_DIM,), jnp.float32),
          pltpu.VMEM((NUM_HP_BINS * EMB_DIM,), jnp.float32),
          pltpu.VMEM((BLOCK + 16,), jnp.int32),
          pltpu.VMEM((BLOCK,), jnp.float32),
          pltpu.VMEM((BLOCK + 16,), jnp.int32),
          pltpu.VMEM((ROWS, u, 2 * EMB_DIM), jnp.float32),
      ],
  )
  def emb_kernel(ut_hbm, hp_hbm, utab_hbm, htab_hbm, out_hbm,
                 utab_v, htab_v, uidx_v, hp_v, hidx_v, cbuf):
    wid = lax.axis_index("s") * NC + lax.axis_index("c")
    kbase_w = wid * rows_w * u
    row_w = wid * rows_w

    pltpu.sync_copy(utab_hbm, utab_v)
    pltpu.sync_copy(htab_hbm, htab_v)

    def block_body(blk, _):
      kb = kbase_w + blk * BLOCK
      pltpu.sync_copy(ut_hbm.at[pl.ds(kb, BLOCK)],
                      uidx_v.at[pl.ds(0, BLOCK)])
      pltpu.sync_copy(hp_hbm.at[pl.ds(kb, BLOCK)], hp_v)

      @plsc.parallel_loop(0, BLOCK // 16, unroll=4)
      def hp_cvt(k):
        v = hp_v[pl.ds(k * 16, 16)] * float(NUM_HP_BINS - 1)
        hidx_v[pl.ds(k * 16, 16)] = v.astype(jnp.int32) << 5

      def sub_body(s, _):
        for r in range(ROWS):
          @plsc.parallel_loop(0, u, unroll=4)
          def lookup(ui, r=r):
            off = s * (ROWS * u) + r * u + ui
            ub = uidx_v[pl.ds(off, 16)][0] << 5
            hb = hidx_v[pl.ds(off, 16)][0]
            cbuf[r, ui, pl.ds(0, 16)] = utab_v[pl.ds(ub, 16)]
            cbuf[r, ui, pl.ds(16, 16)] = utab_v[pl.ds(ub + 16, 16)]
            cbuf[r, ui, pl.ds(32, 16)] = htab_v[pl.ds(hb, 16)]
            cbuf[r, ui, pl.ds(48, 16)] = htab_v[pl.ds(hb + 16, 16)]

        row0 = row_w + blk * rows_blk + s * ROWS
        pltpu.sync_copy(cbuf, out_hbm.at[pl.ds(row0, ROWS)])
        return ()

      lax.fori_loop(0, n_sub, sub_body, ())
      return ()

    lax.fori_loop(0, n_blocks, block_body, ())

  return emb_kernel


def kernel(utype, hp, utype_table, hp_table):
  b, u = utype.shape
  n_total = b * u
  n_utype = utype_table.shape[0]
  return _make_kernel(b, u, n_utype)(
      utype.reshape(n_total).astype(jnp.int32), hp.reshape(n_total),
      utype_table.reshape(n_utype * EMB_DIM),
      hp_table.reshape(NUM_HP_BINS * EMB_DIM))


# speedup vs baseline: 11.9415x; 1.1415x over previous
import functools

import jax
import jax.numpy as jnp
from jax import lax
from jax.experimental import pallas as pl
from jax.experimental.pallas import tpu as pltpu
from jax.experimental.pallas import tpu_sc as plsc

EMB_DIM = 32
NUM_HP_BINS = 256

NC = 2
NS = 16
NW = NC * NS

BLOCK = 3200


def _make_kernel(b, u, n_utype):
  rows_w = b // NW
  n_blocks = (rows_w * u) // BLOCK
  rows_blk = BLOCK // u
  mesh = plsc.VectorSubcoreMesh(
      core_axis_name="c", subcore_axis_name="s", num_cores=NC, num_subcores=NS
  )

  @functools.partial(
      pl.kernel,
      out_type=jax.ShapeDtypeStruct((b, u, 2 * EMB_DIM), jnp.float32),
      mesh=mesh,
      scratch_types=[
          pltpu.VMEM((n_utype * EMB_DIM,), jnp.float32),
          pltpu.VMEM((NUM_HP_BINS * EMB_DIM,), jnp.float32),
          pltpu.VMEM((BLOCK + 16,), jnp.int32),
          pltpu.VMEM((BLOCK,), jnp.float32),
          pltpu.VMEM((BLOCK + 16,), jnp.int32),
          pltpu.VMEM((u, 2 * EMB_DIM), jnp.float32),
          pltpu.VMEM((u, 2 * EMB_DIM), jnp.float32),
          pltpu.SemaphoreType.DMA,
          pltpu.SemaphoreType.DMA,
      ],
  )
  def emb_kernel(ut_hbm, hp_hbm, utab_hbm, htab_hbm, out_hbm,
                 utab_v, htab_v, uidx_v, hp_v, hidx_v, cba, cbb, sema, semb):
    wid = lax.axis_index("s") * NC + lax.axis_index("c")
    kbase_w = wid * rows_w * u
    row_w = wid * rows_w
    cbs = (cba, cbb)
    sems = (sema, semb)

    pltpu.sync_copy(utab_hbm, utab_v)
    pltpu.sync_copy(htab_hbm, htab_v)

    def block_body(blk, _):
      kb = kbase_w + blk * BLOCK
      pltpu.sync_copy(ut_hbm.at[pl.ds(kb, BLOCK)],
                      uidx_v.at[pl.ds(0, BLOCK)])
      pltpu.sync_copy(hp_hbm.at[pl.ds(kb, BLOCK)], hp_v)

      @plsc.parallel_loop(0, BLOCK // 16, unroll=4)
      def hp_cvt(k):
        v = hp_v[pl.ds(k * 16, 16)] * float(NUM_HP_BINS - 1)
        hidx_v[pl.ds(k * 16, 16)] = v.astype(jnp.int32) << 5

      copies = [None, None]
      for s in range(rows_blk):
        par = s % 2
        cb = cbs[par]
        if copies[par] is not None:
          copies[par].wait()

        @plsc.parallel_loop(0, u, unroll=4)
        def lookup(ui, s=s, cb=cb):
          off = s * u + ui
          ub = uidx_v[pl.ds(off, 16)][0] << 5
          hb = hidx_v[pl.ds(off, 16)][0]
          cb[ui, pl.ds(0, 16)] = utab_v[pl.ds(ub, 16)]
          cb[ui, pl.ds(16, 16)] = utab_v[pl.ds(ub + 16, 16)]
          cb[ui, pl.ds(32, 16)] = htab_v[pl.ds(hb, 16)]
          cb[ui, pl.ds(48, 16)] = htab_v[pl.ds(hb + 16, 16)]

        row = row_w + blk * rows_blk + s
        copies[par] = pltpu.async_copy(cb, out_hbm.at[row], sems[par])

      copies[0].wait()
      copies[1].wait()
      return ()

    lax.fori_loop(0, n_blocks, block_body, ())

  return emb_kernel


def kernel(utype, hp, utype_table, hp_table):
  b, u = utype.shape
  n_total = b * u
  n_utype = utype_table.shape[0]
  return _make_kernel(b, u, n_utype)(
      utype.reshape(n_total).astype(jnp.int32), hp.reshape(n_total),
      utype_table.reshape(n_utype * EMB_DIM),
      hp_table.reshape(NUM_HP_BINS * EMB_DIM))


# speedup vs baseline: 15.8974x; 1.3313x over previous
import functools

import jax
import jax.numpy as jnp
from jax import lax
from jax.experimental import pallas as pl
from jax.experimental.pallas import tpu as pltpu
from jax.experimental.pallas import tpu_sc as plsc

EMB_DIM = 32
NUM_HP_BINS = 256

NC = 2
NS = 16
NW = NC * NS

TROW = EMB_DIM + 1


def _make_kernel(b, u, n_utype):
  rows_w = b // NW
  n_look = rows_w * u
  n_bg = rows_w // 16
  mesh = plsc.VectorSubcoreMesh(
      core_axis_name="c", subcore_axis_name="s", num_cores=NC, num_subcores=NS
  )

  @functools.partial(
      pl.kernel,
      out_type=jax.ShapeDtypeStruct((u, 2 * EMB_DIM, b), jnp.float32),
      mesh=mesh,
      scratch_types=[
          pltpu.VMEM((n_utype * TROW,), jnp.float32),
          pltpu.VMEM((NUM_HP_BINS * TROW,), jnp.float32),
          pltpu.VMEM((n_look,), jnp.int32),
          pltpu.VMEM((n_look,), jnp.float32),
          pltpu.VMEM((2 * EMB_DIM, 128), jnp.float32),
          pltpu.VMEM((2 * EMB_DIM, 128), jnp.float32),
          pltpu.SemaphoreType.DMA,
          pltpu.SemaphoreType.DMA,
      ],
      compiler_params=pltpu.CompilerParams(needs_layout_passes=False),
  )
  def emb_kernel(ut_hbm, hp_hbm, utab_hbm, htab_hbm, out_hbm,
                 utab_v, htab_v, uidx_v, hp_v, cba, cbb, sema, semb):
    wid = lax.axis_index("s") * NC + lax.axis_index("c")
    kbase = wid * n_look
    b0 = wid * rows_w
    cbs = (cba, cbb)
    sems = (sema, semb)

    pltpu.sync_copy(utab_hbm, utab_v)
    pltpu.sync_copy(htab_hbm, htab_v)
    pltpu.sync_copy(ut_hbm.at[pl.ds(kbase, n_look)], uidx_v)
    pltpu.sync_copy(hp_hbm.at[pl.ds(kbase, n_look)], hp_v)

    l200 = lax.iota(jnp.int32, 16) * u

    def make_unit(cb):
      def unit_body(ui):
        @plsc.parallel_loop(0, n_bg, unroll=2)
        def bg_body(g):
          idxv = l200 + (g * (16 * u) + ui)
          uu = plsc.load_gather(uidx_v, [idxv]) * TROW
          hpv = plsc.load_gather(hp_v, [idxv])
          hh = (hpv * float(NUM_HP_BINS - 1)).astype(jnp.int32) * TROW
          for c in range(EMB_DIM):
            vu = plsc.load_gather(utab_v, [uu + c])
            cb[c, pl.ds(g * 16, 16)] = vu
            vh = plsc.load_gather(htab_v, [hh + c])
            cb[EMB_DIM + c, pl.ds(g * 16, 16)] = vh
      return unit_body

    def fire(ui, par):
      return pltpu.async_copy(
          cbs[par], out_hbm.at[ui, :, pl.ds(b0, 128)], sems[par])

    make_unit(cbs[0])(0)
    cp0 = fire(0, 0)
    make_unit(cbs[1])(1)
    cp1 = fire(1, 1)

    def unit_pair(p, _):
      ui = 2 * p + 2
      cp0.wait()
      make_unit(cbs[0])(ui)
      fire(ui, 0)
      cp1.wait()
      make_unit(cbs[1])(ui + 1)
      fire(ui + 1, 1)
      return ()

    lax.fori_loop(0, (u - 2) // 2, unit_pair, ())
    cp0.wait()
    cp1.wait()

  return emb_kernel


def kernel(utype, hp, utype_table, hp_table):
  b, u = utype.shape
  n_total = b * u
  n_utype = utype_table.shape[0]
  utab_p = jnp.pad(utype_table, ((0, 0), (0, TROW - EMB_DIM))).reshape(-1)
  htab_p = jnp.pad(hp_table, ((0, 0), (0, TROW - EMB_DIM))).reshape(-1)
  out_t = _make_kernel(b, u, n_utype)(
      utype.reshape(n_total).astype(jnp.int32), hp.reshape(n_total),
      utab_p, htab_p)
  return jnp.transpose(out_t, (2, 0, 1))


# speedup vs baseline: 27.9403x; 1.7575x over previous
import functools

import jax
import jax.numpy as jnp
from jax import lax
from jax.experimental import pallas as pl
from jax.experimental.pallas import tpu as pltpu
from jax.experimental.pallas import tpu_sc as plsc

EMB_DIM = 32
NUM_HP_BINS = 256

NC = 2
NS = 16
NW = NC * NS

TROW = EMB_DIM + 1


def _make_kernel(b, u, n_utype):
  rows_w = b // NW
  n_look = rows_w * u
  n_bg = rows_w // 16
  mesh = plsc.VectorSubcoreMesh(
      core_axis_name="c", subcore_axis_name="s", num_cores=NC, num_subcores=NS
  )

  @functools.partial(
      pl.kernel,
      out_type=jax.ShapeDtypeStruct((u, 2 * EMB_DIM, b), jnp.float32),
      mesh=mesh,
      scratch_types=[
          pltpu.VMEM((n_utype * TROW,), jnp.float32),
          pltpu.VMEM((NUM_HP_BINS * TROW,), jnp.float32),
          pltpu.VMEM((n_look,), jnp.int32),
          pltpu.VMEM((n_look,), jnp.float32),
          pltpu.VMEM((2 * EMB_DIM, 128), jnp.float32),
          pltpu.VMEM((2 * EMB_DIM, 128), jnp.float32),
          pltpu.SemaphoreType.DMA,
          pltpu.SemaphoreType.DMA,
      ],
      compiler_params=pltpu.CompilerParams(needs_layout_passes=False),
  )
  def emb_kernel(ut_hbm, hp_hbm, utab_hbm, htab_hbm, out_hbm,
                 utab_v, htab_v, uidx_v, hp_v, cba, cbb, sema, semb):
    wid = lax.axis_index("s") * NC + lax.axis_index("c")
    kbase = wid * n_look
    b0 = wid * rows_w
    cbs = (cba, cbb)
    sems = (sema, semb)

    pltpu.sync_copy(utab_hbm, utab_v)
    pltpu.sync_copy(htab_hbm, htab_v)
    pltpu.sync_copy(ut_hbm.at[pl.ds(kbase, n_look)], uidx_v)
    pltpu.sync_copy(hp_hbm.at[pl.ds(kbase, n_look)], hp_v)

    l200 = lax.iota(jnp.int32, 16) * u

    def make_unit(cb):
      def unit_body(ui):
        @plsc.parallel_loop(0, n_bg, unroll=4)
        def bg_body(g):
          idxv = l200 + (g * (16 * u) + ui)
          uu = plsc.load_gather(uidx_v, [idxv]) * TROW
          hpv = plsc.load_gather(hp_v, [idxv])
          hh = (hpv * float(NUM_HP_BINS - 1)).astype(jnp.int32) * TROW
          for c in range(EMB_DIM):
            vu = plsc.load_gather(utab_v, [uu + c])
            cb[c, pl.ds(g * 16, 16)] = vu
            vh = plsc.load_gather(htab_v, [hh + c])
            cb[EMB_DIM + c, pl.ds(g * 16, 16)] = vh
      return unit_body

    def fire(ui, par):
      return pltpu.async_copy(
          cbs[par], out_hbm.at[ui, :, pl.ds(b0, 128)], sems[par])

    make_unit(cbs[0])(0)
    cp0 = fire(0, 0)
    make_unit(cbs[1])(1)
    cp1 = fire(1, 1)

    def unit_pair(p, _):
      ui = 2 * p + 2
      cp0.wait()
      make_unit(cbs[0])(ui)
      fire(ui, 0)
      cp1.wait()
      make_unit(cbs[1])(ui + 1)
      fire(ui + 1, 1)
      return ()

    lax.fori_loop(0, (u - 2) // 2, unit_pair, ())
    cp0.wait()
    cp1.wait()

  return emb_kernel


def kernel(utype, hp, utype_table, hp_table):
  b, u = utype.shape
  n_total = b * u
  n_utype = utype_table.shape[0]
  utab_p = jnp.pad(utype_table, ((0, 0), (0, TROW - EMB_DIM))).reshape(-1)
  htab_p = jnp.pad(hp_table, ((0, 0), (0, TROW - EMB_DIM))).reshape(-1)
  out_t = _make_kernel(b, u, n_utype)(
      utype.reshape(n_total).astype(jnp.int32), hp.reshape(n_total),
      utab_p, htab_p)
  return jnp.transpose(out_t, (2, 0, 1))
